# baseline (speedup 1.0000x reference)
import functools

import jax
import jax.numpy as jnp
from jax import lax
from jax.experimental import pallas as pl
from jax.experimental.pallas import tpu as pltpu
from jax.experimental.pallas import tpu_sc as plsc

V = 5000
T = 50
TP = 64
B = 32
D = 768
EN = 200000
NPAIR = TP * TP
NCHUNK = 32
HALF = 16


def _sc_body(tpad_hbm, em16_hbm, ew16_hbm, nt_hbm, w_hbm, h_hbm,
             t_v, pr_v, pl_v, er_v, el_v, buf_v, wout_v, hbuf_v,
             semh, semg):
    nc = 2
    b = lax.axis_index("s") * nc + lax.axis_index("c")

    pltpu.sync_copy(tpad_hbm.at[b], t_v)

    hcp = pltpu.async_copy(nt_hbm.at[t_v], hbuf_v, semh)

    iota16 = lax.iota(jnp.int32, 16)

    def u_body(u, carry):
        uvec = jnp.zeros((16,), jnp.int32) + u
        tu = plsc.load_gather(t_v, [uvec])
        tuv = tu * V
        row = u // 2
        col0 = (u % 2) * TP
        for c in range(4):
            tv = t_v[pl.ds(c * 16, 16)]
            p = tuv + tv
            pr_v[row, pl.ds(col0 + c * 16, 16)] = p >> 4
            pl_v[row, pl.ds(col0 + c * 16, 16)] = p & 15
        return carry

    lax.fori_loop(0, TP, u_body, 0)

    for half in range(2):
        jbase = half * HALF

        def fire_em(k, carry):
            pltpu.async_copy(em16_hbm.at[pr_v.at[jbase + k]],
                             buf_v.at[pl.ds(k * 128, 128)], semg)
            return carry

        lax.fori_loop(0, HALF, fire_em, 0)

        def drain_em(k, carry):
            pltpu.make_async_copy(em16_hbm.at[pr_v.at[jbase + k]],
                                  buf_v.at[pl.ds(k * 128, 128)], semg).wait()
            for m in range(8):
                rows = k * 128 + m * 16 + iota16
                lanes = pl_v[jbase + k, pl.ds(m * 16, 16)]
                eid = plsc.load_gather(buf_v, [rows, lanes])
                er_v[jbase + k, pl.ds(m * 16, 16)] = eid >> 4
                el_v[jbase + k, pl.ds(m * 16, 16)] = eid & 15
            return carry

        lax.fori_loop(0, HALF, drain_em, 0)

        def fire_ew(k, carry):
            pltpu.async_copy(ew16_hbm.at[er_v.at[jbase + k]],
                             buf_v.at[pl.ds(k * 128, 128)], semg)
            return carry

        lax.fori_loop(0, HALF, fire_ew, 0)

        def drain_ew(k, carry):
            pltpu.make_async_copy(ew16_hbm.at[er_v.at[jbase + k]],
                                  buf_v.at[pl.ds(k * 128, 128)], semg).wait()
            for m in range(8):
                rows = k * 128 + m * 16 + iota16
                lanes = el_v[jbase + k, pl.ds(m * 16, 16)]
                wi = plsc.load_gather(buf_v, [rows, lanes])
                base = (jbase + k) * 128 + m * 16
                wout_v[pl.ds(base, 16)] = plsc.bitcast(wi, jnp.float32)
            return carry

        lax.fori_loop(0, HALF, drain_ew, 0)

    hcp.wait()
    pltpu.sync_copy(hbuf_v, h_hbm.at[pl.ds(b * TP, TP)])
    pltpu.sync_copy(wout_v, w_hbm.at[b])


@functools.cache
def _make_sc_gather():
  return functools.partial(
    pl.kernel,
    out_type=[
        jax.ShapeDtypeStruct((B, NPAIR), jnp.float32),
        jax.ShapeDtypeStruct((B * TP, D), jnp.float32),
    ],
    mesh=plsc.VectorSubcoreMesh(core_axis_name="c", subcore_axis_name="s"),
    compiler_params=pltpu.CompilerParams(needs_layout_passes=False,
                                         use_tc_tiling_on_sc=False),
    scratch_types=[
        pltpu.VMEM((TP,), jnp.int32),
        pltpu.VMEM((NCHUNK, 128), jnp.int32),
        pltpu.VMEM((NCHUNK, 128), jnp.int32),
        pltpu.VMEM((NCHUNK, 128), jnp.int32),
        pltpu.VMEM((NCHUNK, 128), jnp.int32),
        pltpu.VMEM((HALF * 128, 16), jnp.int32),
        pltpu.VMEM((NPAIR,), jnp.float32),
        pltpu.VMEM((TP, D), jnp.float32),
        pltpu.SemaphoreType.DMA,
        pltpu.SemaphoreType.DMA,
    ],
  )(_sc_body)


def _tc_body(trow_ref, tcol_ref, h_ref, w_ref, eta_ref, W1_ref, b1_ref,
             o_ref):
    trow = trow_ref[0]
    tcol = tcol_ref[0]
    eq = tcol == trow
    isub = lax.broadcasted_iota(jnp.int32, (TP, TP), 0)
    ilane = lax.broadcasted_iota(jnp.int32, (TP, TP), 1)
    one = jnp.ones((TP, TP), jnp.float32)
    zero = jnp.zeros((TP, TP), jnp.float32)
    dupc = jnp.sum(jnp.where(eq & (ilane < isub), one, zero), axis=1,
                   keepdims=True)
    valid_col = (dupc == 0.0) & (tcol != 0)
    dupr = jnp.sum(jnp.where(eq & (isub < ilane), one, zero), axis=0,
                   keepdims=True)
    valid_row = (dupr == 0.0) & (trow != 0)

    h = h_ref[0]
    w = w_ref[0]
    hm = jnp.where(valid_col, h, -1e30)
    wm = jnp.where(valid_col, w, 1.0)

    parts = []
    for c in range(4):
        wv = wm[:, c * 16:(c + 1) * 16]
        cand = hm[:, None, :] * wv[:, :, None]
        parts.append(jnp.max(cand, axis=0))
    h_after = jnp.concatenate(parts, axis=0)

    eta = eta_ref[...]
    new_h = eta * h + (1.0 - eta) * h_after
    vcf = jnp.where(valid_col, 1.0, 0.0)
    cnt = jnp.maximum(jnp.sum(jnp.where(valid_row, 1.0, 0.0)), 1.0)
    g = jnp.sum(new_h * vcf, axis=0, keepdims=True) / cnt

    z = lax.dot_general(g, W1_ref[...], (((1,), (1,)), ((), ())),
                        preferred_element_type=jnp.float32) + b1_ref[...]
    o_ref[0] = 1.0 / (1.0 + jnp.exp(-z))


def kernel(token_ids, node_table, edge_weights, edge_matrix, node_eta,
           W1, b1):
    t32 = token_ids.astype(jnp.int32)
    tpad = jnp.zeros((B, TP), jnp.int32).at[:, :T].set(t32)
    em16 = edge_matrix.reshape(V * V // 16, 16)
    ew16 = lax.bitcast_convert_type(edge_weights.reshape(EN // 16, 16),
                                    jnp.int32)

    w_flat, h_flat = _make_sc_gather()(tpad, em16, ew16, node_table)

    h3 = h_flat.reshape(B, TP, D)
    w3 = w_flat.reshape(B, TP, TP)
    trow = tpad.reshape(B, 1, TP)
    tcol = tpad.reshape(B, TP, 1)
    eta2 = node_eta.reshape(1, 1)
    b12 = b1.reshape(1, D)

    out3 = pl.pallas_call(
        _tc_body,
        grid=(B,),
        in_specs=[
            pl.BlockSpec((1, 1, TP), lambda b: (b, 0, 0)),
            pl.BlockSpec((1, TP, 1), lambda b: (b, 0, 0)),
            pl.BlockSpec((1, TP, D), lambda b: (b, 0, 0)),
            pl.BlockSpec((1, TP, TP), lambda b: (b, 0, 0)),
            pl.BlockSpec((1, 1), lambda b: (0, 0)),
            pl.BlockSpec((D, D), lambda b: (0, 0)),
            pl.BlockSpec((1, D), lambda b: (0, 0)),
        ],
        out_specs=pl.BlockSpec((1, 1, D), lambda b: (b, 0, 0)),
        out_shape=jax.ShapeDtypeStruct((B, 1, D), jnp.float32),
    )(trow, tcol, h3, w3, eta2, W1, b12)

    return out3.reshape(B, D)


# speedup vs baseline: 1.3465x; 1.3465x over previous
import functools

import jax
import jax.numpy as jnp
from jax import lax
from jax.experimental import pallas as pl
from jax.experimental.pallas import tpu as pltpu
from jax.experimental.pallas import tpu_sc as plsc

V = 5000
T = 50
TP = 64
B = 32
D = 768
EN = 200000
TV = 56
NPAIR = TV * T
NCHUNK = 25
CHW = 112

_SC_PARAMS = pltpu.CompilerParams(needs_layout_passes=False,
                                  use_tc_tiling_on_sc=False)


def _worker_id():
    return lax.axis_index("s") * 2 + lax.axis_index("c")


def _sc_h_body(tpad_hbm, nt_hbm, h_hbm, t_v, hbuf_v, sem):
    b = _worker_id()
    pltpu.sync_copy(tpad_hbm.at[b], t_v)
    pltpu.async_copy(nt_hbm.at[t_v], hbuf_v, sem).wait()
    pltpu.sync_copy(hbuf_v, h_hbm.at[pl.ds(b * TP, TP)])


def _sc_w_body(tpad_hbm, em_hbm, ew_hbm, w_hbm,
               t_v, pidx_v, eid_v, wbuf_v, semE, semW):
    b = _worker_id()
    pltpu.sync_copy(tpad_hbm.at[b], t_v)

    iota16 = lax.iota(jnp.int32, 16)

    def build_fire(k, carry):
        for m in range(7):
            i = k * CHW + m * 16 + iota16
            v = (i * 1311) >> 16
            u = i - v * 50
            tu = plsc.load_gather(t_v, [u])
            tv = plsc.load_gather(t_v, [v])
            pidx_v[k, pl.ds(m * 16, 16)] = tu * V + tv
        pltpu.async_copy(em_hbm.at[pidx_v.at[k]], eid_v.at[k], semE)
        return carry

    lax.fori_loop(0, NCHUNK, build_fire, 0)

    def drain_fire(k, carry):
        pltpu.make_async_copy(em_hbm.at[pidx_v.at[k]], eid_v.at[k],
                              semE).wait()
        pltpu.async_copy(ew_hbm.at[eid_v.at[k]], wbuf_v.at[k], semW)
        return carry

    lax.fori_loop(0, NCHUNK, drain_fire, 0)

    def drain_w(k, carry):
        pltpu.make_async_copy(ew_hbm.at[eid_v.at[k]], wbuf_v.at[k],
                              semW).wait()
        return carry

    lax.fori_loop(0, NCHUNK, drain_w, 0)

    pltpu.sync_copy(wbuf_v, w_hbm.at[b])


@functools.cache
def _make_sc_kernels():
  mesh = plsc.VectorSubcoreMesh(core_axis_name="c", subcore_axis_name="s")
  sc_h = functools.partial(
    pl.kernel,
    out_type=jax.ShapeDtypeStruct((B * TP, D), jnp.float32),
    mesh=mesh,
    compiler_params=_SC_PARAMS,
    scratch_types=[
        pltpu.VMEM((TP,), jnp.int32),
        pltpu.VMEM((TP, D), jnp.float32),
        pltpu.SemaphoreType.DMA,
    ],
  )(_sc_h_body)
  sc_w = functools.partial(
    pl.kernel,
    out_type=jax.ShapeDtypeStruct((B, NCHUNK, CHW), jnp.float32),
    mesh=mesh,
    compiler_params=_SC_PARAMS,
    scratch_types=[
        pltpu.VMEM((TP,), jnp.int32),
        pltpu.VMEM((NCHUNK, CHW), jnp.int32),
        pltpu.VMEM((NCHUNK, CHW), jnp.int32),
        pltpu.VMEM((NCHUNK, CHW), jnp.float32),
        pltpu.SemaphoreType.DMA,
        pltpu.SemaphoreType.DMA,
    ],
  )(_sc_w_body)
  return sc_h, sc_w


def _tc_body(trow_ref, tcol_ref, h_ref, wt_ref, eta_ref, W1_ref, b1_ref,
             o_ref):
    trow = trow_ref[0]
    tcol = tcol_ref[0]
    eq = tcol == trow
    isub = lax.broadcasted_iota(jnp.int32, (TP, TP), 0)
    ilane = lax.broadcasted_iota(jnp.int32, (TP, TP), 1)
    one = jnp.ones((TP, TP), jnp.float32)
    zero = jnp.zeros((TP, TP), jnp.float32)
    dupc = jnp.sum(jnp.where(eq & (ilane < isub), one, zero), axis=1,
                   keepdims=True)
    valid_col = (dupc == 0.0) & (tcol != 0)
    dupr = jnp.sum(jnp.where(eq & (isub < ilane), one, zero), axis=0,
                   keepdims=True)
    valid_row = (dupr == 0.0) & (trow != 0)

    h = h_ref[0]
    wt = wt_ref[0]
    hm = jnp.where(valid_col, h, -1e30)
    wtm = jnp.where(valid_row[:, :T], wt, 1.0)

    h_after = jnp.full((TV, D), -1e30, jnp.float32)
    for u in range(T):
        cand = wtm[:, u:u + 1] * hm[u:u + 1, :]
        h_after = jnp.maximum(h_after, cand)

    eta = eta_ref[...]
    new_h = eta * h[:TV, :] + (1.0 - eta) * h_after
    vcf = jnp.where(valid_col[:TV, :], 1.0, 0.0)
    cnt = jnp.maximum(jnp.sum(jnp.where(valid_row, 1.0, 0.0)), 1.0)
    g = jnp.sum(new_h * vcf, axis=0, keepdims=True) / cnt

    z = lax.dot_general(g, W1_ref[...], (((1,), (1,)), ((), ())),
                        preferred_element_type=jnp.float32) + b1_ref[...]
    o_ref[0] = 1.0 / (1.0 + jnp.exp(-z))


def kernel(token_ids, node_table, edge_weights, edge_matrix, node_eta,
           W1, b1):
    t32 = token_ids.astype(jnp.int32)
    tpad = jnp.zeros((B, TP), jnp.int32).at[:, :T].set(t32)
    em_flat = edge_matrix.reshape(V * V)
    ew_flat = edge_weights.reshape(EN)

    sc_h, sc_w = _make_sc_kernels()
    h_flat = sc_h(tpad, node_table)
    w_raw = sc_w(tpad, em_flat, ew_flat)

    h3 = h_flat.reshape(B, TP, D)
    wt3 = w_raw.reshape(B, TV, T)
    trow = tpad.reshape(B, 1, TP)
    tcol = tpad.reshape(B, TP, 1)
    eta2 = node_eta.reshape(1, 1)
    b12 = b1.reshape(1, D)

    out3 = pl.pallas_call(
        _tc_body,
        grid=(B,),
        in_specs=[
            pl.BlockSpec((1, 1, TP), lambda b: (b, 0, 0)),
            pl.BlockSpec((1, TP, 1), lambda b: (b, 0, 0)),
            pl.BlockSpec((1, TP, D), lambda b: (b, 0, 0)),
            pl.BlockSpec((1, TV, T), lambda b: (b, 0, 0)),
            pl.BlockSpec((1, 1), lambda b: (0, 0)),
            pl.BlockSpec((D, D), lambda b: (0, 0)),
            pl.BlockSpec((1, D), lambda b: (0, 0)),
        ],
        out_specs=pl.BlockSpec((1, 1, D), lambda b: (b, 0, 0)),
        out_shape=jax.ShapeDtypeStruct((B, 1, D), jnp.float32),
    )(trow, tcol, h3, wt3, eta2, W1, b12)

    return out3.reshape(B, D)
